# Initial kernel scaffold; baseline (speedup 1.0000x reference)
import jax
import jax.numpy as jnp
from jax.experimental import pallas as pl


def kernel(coords, features, times, batch_ids, params):
    raise NotImplementedError("write your pallas kernel here")


# speedup vs baseline: 1.9879x; 1.9879x over previous
import functools

import jax
import jax.numpy as jnp
from jax import lax
from jax.experimental import pallas as pl
from jax.experimental.pallas import tpu as pltpu
from jax.experimental.pallas import tpu_sc as plsc

def _bdot(a, b):
    return jnp.dot(a.astype(jnp.bfloat16), b.astype(jnp.bfloat16),
                   preferred_element_type=jnp.float32)


N = 131072
NB = 16
K = 128
BLK = 2048
R = 1024
C = 128
TOT = NB * K


def _mlp_scores_body(c_ref, t_ref, f_ref, lng, lnb, w1, b1, w2, b2,
                     w0a, w0b, b0, wf, bf, hw1, hb1, hw2r, hb2,
                     s_ref, comb_ref):
    xyzt = jnp.concatenate([c_ref[:], t_ref[:]], axis=1)
    mu = jnp.mean(xyzt, axis=-1, keepdims=True)
    var = jnp.mean((xyzt - mu) ** 2, axis=-1, keepdims=True)
    xn = (xyzt - mu) * lax.rsqrt(var + 1e-5) * lng[:] + lnb[:]
    h = jax.nn.relu(_bdot(xn, w1[:]) + b1[:])
    h = jax.nn.relu(_bdot(h, w2[:]) + b2[:])
    z = jax.nn.relu(_bdot(f_ref[:], w0a[:])
                    + _bdot(h, w0b[:]) + b0[:])
    pf = _bdot(z, wf[:]) + bf[:]
    g = jax.nn.relu(_bdot(pf, hw1[:]) + hb1[:])
    g16 = g.astype(jnp.bfloat16).astype(jnp.float32)
    v16 = hw2r[:].astype(jnp.bfloat16).astype(jnp.float32)
    s_ref[:] = jnp.sum(g16 * v16, axis=1, keepdims=True) + hb2[0, 0]
    comb_ref[:] = jnp.concatenate(
        [f_ref[:], xyzt, jnp.zeros((BLK, 60), jnp.float32)], axis=1)


def _topk_body(s_ref, ids_ref, sel_ref, msk_ref, keyed_ref):
    b = pl.program_id(0)
    ids = ids_ref[:]
    inb = ids == b
    cnt = jnp.sum(inb.astype(jnp.int32))
    neginf = jnp.float32(-jnp.inf)
    keyed_ref[:] = jnp.where(inb, s_ref[:], neginf)
    rows = lax.broadcasted_iota(jnp.int32, (R, C), 0)
    cols = lax.broadcasted_iota(jnp.int32, (R, C), 1)
    flat = rows * C + cols
    lane = lax.broadcasted_iota(jnp.int32, (1, K), 1)

    def step(t, selrow):
        kd = keyed_ref[:]
        m = jnp.max(kd)
        idx = jnp.min(jnp.where(kd == m, flat, jnp.int32(N)))
        keyed_ref[:] = jnp.where(flat == idx, neginf, kd)
        return jnp.where(lane == t, idx, selrow)

    selrow = lax.fori_loop(0, K, step, jnp.full((1, K), N - 1, jnp.int32))
    valid = lane < cnt
    sel_ref[0] = jnp.where(valid, jnp.minimum(selrow, N - 1), N - 1)
    msk_ref[0] = valid.astype(jnp.int32)


def _tokens_body(c_ref, m_ref, lng, lnb, w1, b1, w2, b2,
                 w0a, w0b, b0, wf, bf, tok_ref, cen_ref):
    f_sel = c_ref[:, :64]
    xyzt = c_ref[:, 64:68]
    mu = jnp.mean(xyzt, axis=-1, keepdims=True)
    var = jnp.mean((xyzt - mu) ** 2, axis=-1, keepdims=True)
    xn = (xyzt - mu) * lax.rsqrt(var + 1e-5) * lng[:] + lnb[:]
    h = jax.nn.relu(_bdot(xn, w1[:]) + b1[:])
    h = jax.nn.relu(_bdot(h, w2[:]) + b2[:])
    z = jax.nn.relu(_bdot(f_sel, w0a[:])
                    + _bdot(h, w0b[:]) + b0[:])
    pf = _bdot(z, wf[:]) + bf[:]
    mk = m_ref[:]
    tok_ref[:] = pf * mk
    cen_ref[:] = xyzt * mk


def _full(shape):
    return pl.BlockSpec(shape, lambda *_: tuple(0 for _ in shape))


def _scores_call(coords, times, features, ws):
    grid = (N // BLK,)
    row = lambda i: (i, 0)
    in_specs = [
        pl.BlockSpec((BLK, 3), row),
        pl.BlockSpec((BLK, 1), row),
        pl.BlockSpec((BLK, 64), row),
    ] + [_full(w.shape) for w in ws]
    return pl.pallas_call(
        _mlp_scores_body,
        grid=grid,
        in_specs=in_specs,
        out_specs=[pl.BlockSpec((BLK, 1), row), pl.BlockSpec((BLK, 128), row)],
        out_shape=[jax.ShapeDtypeStruct((N, 1), jnp.float32),
                   jax.ShapeDtypeStruct((N, 128), jnp.float32)],
    )(coords, times, features, *ws)


def _topk_call(s2d, ids2d):
    return pl.pallas_call(
        _topk_body,
        grid=(NB,),
        in_specs=[_full((R, C)), _full((R, C))],
        out_specs=[pl.BlockSpec((1, 1, K), lambda b: (b, 0, 0)),
                   pl.BlockSpec((1, 1, K), lambda b: (b, 0, 0))],
        out_shape=[jax.ShapeDtypeStruct((NB, 1, K), jnp.int32),
                   jax.ShapeDtypeStruct((NB, 1, K), jnp.int32)],
        scratch_shapes=[pltpu.VMEM((R, C), jnp.float32)],
    )(s2d, ids2d)


def _sc_gather(comb, idx):
    info = plsc.get_sparse_core_info()
    nw = info.num_cores * info.num_subcores
    bw = TOT // nw
    mesh = plsc.VectorSubcoreMesh(core_axis_name="c", subcore_axis_name="s")

    @functools.partial(
        pl.kernel, mesh=mesh,
        out_type=jax.ShapeDtypeStruct((TOT, 128), jnp.float32),
        scratch_types=[pltpu.VMEM((bw,), jnp.int32),
                       pltpu.VMEM((bw, 128), jnp.float32),
                       pltpu.SemaphoreType.DMA],
    )
    def gather_k(c_hbm, idx_hbm, out_hbm, idx_v, rows_v, sem):
        wid = lax.axis_index("s") * info.num_cores + lax.axis_index("c")
        base = wid * bw
        pltpu.sync_copy(idx_hbm.at[pl.ds(base, bw)], idx_v)
        pltpu.async_copy(c_hbm.at[idx_v], rows_v, sem).wait()
        pltpu.sync_copy(rows_v, out_hbm.at[pl.ds(base, bw)])

    return gather_k(comb, idx)


def _tokens_call(comb_sel, mflat, ws):
    return pl.pallas_call(
        _tokens_body,
        out_shape=[jax.ShapeDtypeStruct((TOT, 128), jnp.float32),
                   jax.ShapeDtypeStruct((TOT, 4), jnp.float32)],
    )(comb_sel, mflat, *ws)


def kernel(coords, features, times, batch_ids, params):
    p = params
    lng = p["ln_g"].reshape(1, 4)
    lnb = p["ln_b"].reshape(1, 4)
    w1 = p["se_W1"]
    b1 = p["se_b1"].reshape(1, 64)
    w2 = p["se_W2"]
    b2 = p["se_b2"].reshape(1, 64)
    w0a = p["mlp_W0"][:64]
    w0b = p["mlp_W0"][64:]
    b0 = p["mlp_b0"].reshape(1, 128)
    wf = p["mlp_Wf"]
    bf = p["mlp_bf"].reshape(1, 128)
    hw1 = p["h_W1"]
    hb1 = p["h_b1"].reshape(1, 64)
    hw2r = p["h_W2"].reshape(1, 64)
    hb2 = p["h_b2"].reshape(1, 1)

    score_ws = [lng, lnb, w1, b1, w2, b2, w0a, w0b, b0, wf, bf, hw1, hb1, hw2r, hb2]
    scores, comb = _scores_call(coords, times, features, score_ws)

    s2d = scores.reshape(R, C)
    ids2d = batch_ids.astype(jnp.int32).reshape(R, C)
    sel, msk = _topk_call(s2d, ids2d)

    comb_sel = _sc_gather(comb, sel.reshape(TOT))

    mflat = msk.reshape(TOT, 1).astype(jnp.float32)
    tok_ws = [lng, lnb, w1, b1, w2, b2, w0a, w0b, b0, wf, bf]
    tok, cen = _tokens_call(comb_sel, mflat, tok_ws)

    return (tok.reshape(NB, K, 128), cen.reshape(NB, K, 4),
            msk.reshape(NB, K).astype(jnp.bool_))
